# baseline (speedup 1.0000x reference)
import functools

import jax
import jax.numpy as jnp
from jax import lax
from jax.experimental import pallas as pl
from jax.experimental.pallas import tpu as pltpu
from jax.experimental.pallas import tpu_sc as plsc

D_MODEL = 768
LANES = 16
NCORES = 2
NSUB = 16
NW = NCORES * NSUB
CHUNK = 64


@functools.partial(jax.jit, static_argnames=("rows", "rpw"))
def _sc_add_pe(x_flat, idx_flat, pe_flat, rows, rpw):
    nchunk = rpw // CHUNK
    mesh = plsc.VectorSubcoreMesh(core_axis_name="c", subcore_axis_name="s")

    @functools.partial(
        pl.kernel,
        out_type=jax.ShapeDtypeStruct((rows * D_MODEL,), jnp.float32),
        mesh=mesh,
        scratch_types=[
            pltpu.VMEM((pe_flat.shape[0],), jnp.float32),
            pltpu.VMEM((rpw,), jnp.int32),
            pltpu.VMEM((CHUNK * D_MODEL,), jnp.float32),
        ],
        compiler_params=pltpu.CompilerParams(needs_layout_passes=False),
    )
    def k(x_hbm, idx_hbm, pe_hbm, out_hbm, pe_v, idx_v, xbuf):
        wid = lax.axis_index("s") * NCORES + lax.axis_index("c")
        row0 = wid * rpw
        pltpu.sync_copy(pe_hbm, pe_v)
        pltpu.sync_copy(idx_hbm.at[pl.ds(row0, rpw)], idx_v)
        iota = lax.iota(jnp.int32, LANES)

        def chunk_body(ci, _):
            e0 = (row0 + ci * CHUNK) * D_MODEL
            pltpu.sync_copy(x_hbm.at[pl.ds(e0, CHUNK * D_MODEL)], xbuf)

            def row_body(r, _):
                rsplat = plsc.load_gather(
                    idx_v, [jnp.zeros((LANES,), jnp.int32) + (ci * CHUNK + r)]
                )
                pbase = rsplat * D_MODEL + iota
                xoff = r * D_MODEL
                for c in range(D_MODEL // LANES):
                    pv = plsc.load_gather(pe_v, [pbase + (c * LANES)])
                    xv = xbuf[pl.ds(xoff + c * LANES, LANES)]
                    xbuf[pl.ds(xoff + c * LANES, LANES)] = xv + pv
                return 0

            lax.fori_loop(0, CHUNK, row_body, 0)
            pltpu.sync_copy(xbuf, out_hbm.at[pl.ds(e0, CHUNK * D_MODEL)])
            return 0

        lax.fori_loop(0, nchunk, chunk_body, 0)

    return k(x_flat, idx_flat, pe_flat)


def kernel(x, created_list, pe):
    b, s, d = x.shape
    rows = b * s
    x_flat = x.reshape(rows * d)
    idx = created_list.reshape(rows).astype(jnp.int32)
    pe_flat = pe.reshape(-1)
    out = _sc_add_pe(x_flat, idx, pe_flat, rows=rows, rpw=rows // NW)
    return out.reshape(b, s, d)
